# baseline (speedup 1.0000x reference)
import jax
import jax.numpy as jnp
from jax.experimental import pallas as pl

_TAU = 0.5
_ROWS = 8
_NS = 4


def _part(l, u):
    m = jnp.max(l, axis=-1, keepdims=True)
    e = jnp.exp((1.0 / _TAU) * (l - m))
    r = 1.0 / jnp.log(u)
    r2 = r * r
    z = e[:, None, :] * r2
    s = jnp.sum(z, axis=-1, keepdims=True)
    return jnp.max(z * (1.0 / s), axis=1)


def _body(*refs):
    logits_ref = refs[0]
    out_ref = refs[-1]
    rows = _ROWS
    for j in range(_NS):
        l = logits_ref[j * rows:(j + 1) * rows, :]
        out_ref[j * rows:(j + 1) * rows, :] = _part(l, refs[1 + j][...])


@jax.jit
def kernel(logits, uniform):
    b, d = logits.shape
    _, k, _ = uniform.shape
    rows = _ROWS
    grid = (b // (rows * _NS),)

    def mk(j):
        return pl.BlockSpec((rows, k, d), lambda i, j=j: (_NS * i + j, 0, 0))

    return pl.pallas_call(
        _body,
        grid=grid,
        in_specs=[pl.BlockSpec((rows * _NS, d), lambda i: (i, 0))]
        + [mk(j) for j in range(_NS)],
        out_specs=pl.BlockSpec((rows * _NS, d), lambda i: (i, 0)),
        out_shape=jax.ShapeDtypeStruct((b, d), jnp.float32),
    )(logits, *([uniform] * _NS))


# speedup vs baseline: 1.0417x; 1.0417x over previous
import jax
import jax.numpy as jnp
from jax.experimental import pallas as pl

_TAU = 0.5
_ROWS = 8
_NS = 2


def _part(l, u):
    m = jnp.max(l, axis=-1, keepdims=True)
    e = jnp.exp((1.0 / _TAU) * (l - m))
    r = 1.0 / jnp.log(u)
    r2 = r * r
    z = e[:, None, :] * r2
    s = jnp.sum(z, axis=-1, keepdims=True)
    return jnp.max(z * (1.0 / s), axis=1)


def _body(*refs):
    logits_ref = refs[0]
    out_ref = refs[-1]
    rows = _ROWS
    for j in range(_NS):
        l = logits_ref[j * rows:(j + 1) * rows, :]
        out_ref[j * rows:(j + 1) * rows, :] = _part(l, refs[1 + j][...])


@jax.jit
def kernel(logits, uniform):
    b, d = logits.shape
    _, k, _ = uniform.shape
    rows = _ROWS
    grid = (b // (rows * _NS),)

    def mk(j):
        return pl.BlockSpec((rows, k, d), lambda i, j=j: (_NS * i + j, 0, 0))

    return pl.pallas_call(
        _body,
        grid=grid,
        in_specs=[pl.BlockSpec((rows * _NS, d), lambda i: (i, 0))]
        + [mk(j) for j in range(_NS)],
        out_specs=pl.BlockSpec((rows * _NS, d), lambda i: (i, 0)),
        out_shape=jax.ShapeDtypeStruct((b, d), jnp.float32),
    )(logits, *([uniform] * _NS))
